# baseline (speedup 1.0000x reference)
import functools

import jax
import jax.numpy as jnp
from jax import lax
from jax.experimental import pallas as pl
from jax.experimental.pallas import tpu as pltpu
from jax.experimental.pallas import tpu_sc as plsc

_LANES = 16
_DEPTH = 12


@functools.lru_cache(maxsize=None)
def _make_sc_gather_t(B, V, D):
    info = plsc.get_sparse_core_info()
    num_workers = info.num_cores * info.num_subcores
    b_per_w = B // num_workers
    n_groups = b_per_w // _LANES
    mesh = plsc.VectorSubcoreMesh(core_axis_name="c", subcore_axis_name="s")

    @functools.partial(
        pl.kernel,
        mesh=mesh,
        out_type=jax.ShapeDtypeStruct((D, B), jnp.float32),
        compiler_params=pltpu.CompilerParams(
            use_tc_tiling_on_sc=True, needs_layout_passes=False
        ),
        scratch_types=[
            pltpu.VMEM((b_per_w,), jnp.int32),
            pltpu.VMEM((_DEPTH, D, 128), jnp.float32),
            pltpu.VMEM((D, b_per_w), jnp.float32),
        ]
        + [pltpu.SemaphoreType.DMA] * _DEPTH,
    )
    def gather_kernel(table_t_hbm, idx_hbm, out_hbm, idx_v, fbuf, stage_v, *sems):
        wid = lax.axis_index("s") * info.num_cores + lax.axis_index("c")
        base = wid * b_per_w
        pltpu.sync_copy(idx_hbm.at[pl.ds(base, b_per_w)], idx_v)
        d_lo = lax.iota(jnp.int32, _LANES)
        d_hi = d_lo + _LANES

        def fetch(xj, slot):
            col0 = pl.multiple_of((xj >> 7) << 7, 128)
            return pltpu.async_copy(
                table_t_hbm.at[:, pl.ds(col0, 128)], fbuf.at[slot], sems[slot]
            )

        def body(g, _):
            vecx = idx_v[pl.ds(g * _LANES, _LANES)]
            copies = {}
            for k in range(_DEPTH - 1):
                copies[k] = fetch(vecx[k], k % _DEPTH)
            for k in range(_LANES):
                if k + _DEPTH - 1 < _LANES:
                    kk = k + _DEPTH - 1
                    copies[kk] = fetch(vecx[kk], kk % _DEPTH)
                copies[k].wait()
                xj = vecx[k]
                c_sp = jnp.full((_LANES,), xj & 127, jnp.int32)
                i_sp = jnp.full((_LANES,), g * _LANES + k, jnp.int32)
                fb = fbuf.at[k % _DEPTH]
                plsc.store_scatter(
                    stage_v, [d_lo, i_sp], plsc.load_gather(fb, [d_lo, c_sp])
                )
                plsc.store_scatter(
                    stage_v, [d_hi, i_sp], plsc.load_gather(fb, [d_hi, c_sp])
                )
            return _

        lax.fori_loop(0, n_groups, body, None)
        pltpu.sync_copy(stage_v, out_hbm.at[:, pl.ds(base, b_per_w)])

    return gather_kernel


def _tc_linear_t_body(h_ref, wmt_ref, bm_ref, wlt_ref, bl_ref, mu_ref, lv_ref):
    hb = h_ref[...]
    mu_ref[...] = (
        jnp.dot(wmt_ref[...], hb, preferred_element_type=jnp.float32) + bm_ref[...]
    )
    lv_ref[...] = (
        jnp.dot(wlt_ref[...], hb, preferred_element_type=jnp.float32) + bl_ref[...]
    )


@functools.lru_cache(maxsize=None)
def _make_tc_linear_t(B, D, O, grid):
    blk = B // grid
    return pl.pallas_call(
        _tc_linear_t_body,
        grid=(grid,),
        in_specs=[
            pl.BlockSpec((D, blk), lambda i: (0, i)),
            pl.BlockSpec((O, D), lambda i: (0, 0)),
            pl.BlockSpec((O, 1), lambda i: (0, 0)),
            pl.BlockSpec((O, D), lambda i: (0, 0)),
            pl.BlockSpec((O, 1), lambda i: (0, 0)),
        ],
        out_specs=[
            pl.BlockSpec((O, blk), lambda i: (0, i)),
            pl.BlockSpec((O, blk), lambda i: (0, i)),
        ],
        out_shape=[
            jax.ShapeDtypeStruct((O, B), jnp.float32),
            jax.ShapeDtypeStruct((O, B), jnp.float32),
        ],
    )


def kernel(x, table, W_mu, b_mu, W_logvar, b_logvar):
    B = x.shape[0]
    V, D = table.shape
    O = W_mu.shape[1]
    h_t = _make_sc_gather_t(B, V, D)(table.T, x)
    mu_t, lv_t = _make_tc_linear_t(B, D, O, 8)(
        h_t, W_mu.T, b_mu.reshape(O, 1), W_logvar.T, b_logvar.reshape(O, 1)
    )
    return (mu_t.T, lv_t.T)


# speedup vs baseline: 1.1158x; 1.1158x over previous
import functools

import jax
import jax.numpy as jnp
from jax import lax
from jax.experimental import pallas as pl
from jax.experimental.pallas import tpu as pltpu
from jax.experimental.pallas import tpu_sc as plsc

_LANES = 16
_DEPTH = 16


@functools.lru_cache(maxsize=None)
def _make_sc_gather_t(B, V, D):
    info = plsc.get_sparse_core_info()
    num_workers = info.num_cores * info.num_subcores
    b_per_w = B // num_workers
    n_groups = b_per_w // _LANES
    mesh = plsc.VectorSubcoreMesh(core_axis_name="c", subcore_axis_name="s")

    @functools.partial(
        pl.kernel,
        mesh=mesh,
        out_type=jax.ShapeDtypeStruct((D, B), jnp.float32),
        compiler_params=pltpu.CompilerParams(
            use_tc_tiling_on_sc=True, needs_layout_passes=False
        ),
        scratch_types=[
            pltpu.VMEM((b_per_w,), jnp.int32),
            pltpu.VMEM((_DEPTH, D, 128), jnp.float32),
            pltpu.VMEM((D, b_per_w), jnp.float32),
        ]
        + [pltpu.SemaphoreType.DMA] * _DEPTH,
    )
    def gather_kernel(table_t_hbm, idx_hbm, out_hbm, idx_v, fbuf, stage_v, *sems):
        wid = lax.axis_index("s") * info.num_cores + lax.axis_index("c")
        base = wid * b_per_w
        pltpu.sync_copy(idx_hbm.at[pl.ds(base, b_per_w)], idx_v)
        d_lo = lax.iota(jnp.int32, _LANES)
        d_hi = d_lo + _LANES

        def fetch(xj, slot):
            col0 = pl.multiple_of((xj >> 7) << 7, 128)
            return pltpu.async_copy(
                table_t_hbm.at[:, pl.ds(col0, 128)], fbuf.at[slot], sems[slot]
            )

        def wait_slot(slot):
            pltpu.make_async_copy(
                table_t_hbm.at[:, pl.ds(0, 128)], fbuf.at[slot], sems[slot]
            ).wait()

        vec0 = idx_v[pl.ds(0, _LANES)]
        for k in range(_DEPTH - 1):
            fetch(vec0[k], k % _DEPTH)

        def body(g, _):
            vec_cur = idx_v[pl.ds(g * _LANES, _LANES)]
            g1 = jnp.minimum(g + 1, n_groups - 1)
            vec_nxt = idx_v[pl.ds(g1 * _LANES, _LANES)]
            for k in range(_LANES):
                ahead = k + _DEPTH - 1
                xa = vec_cur[ahead] if ahead < _LANES else vec_nxt[ahead - _LANES]
                fetch(xa, ahead % _DEPTH)
                wait_slot(k % _DEPTH)
                xj = vec_cur[k]
                c_sp = jnp.full((_LANES,), xj & 127, jnp.int32)
                i_sp = jnp.full((_LANES,), g * _LANES + k, jnp.int32)
                fb = fbuf.at[k % _DEPTH]
                plsc.store_scatter(
                    stage_v, [d_lo, i_sp], plsc.load_gather(fb, [d_lo, c_sp])
                )
                plsc.store_scatter(
                    stage_v, [d_hi, i_sp], plsc.load_gather(fb, [d_hi, c_sp])
                )
            return _

        lax.fori_loop(0, n_groups, body, None)
        for j in range(_DEPTH - 1):
            wait_slot(j % _DEPTH)
        pltpu.sync_copy(stage_v, out_hbm.at[:, pl.ds(base, b_per_w)])

    return gather_kernel


def _tc_linear_t_body(h_ref, wmt_ref, bm_ref, wlt_ref, bl_ref, mu_ref, lv_ref):
    hb = h_ref[...]
    mu_ref[...] = (
        jnp.dot(wmt_ref[...], hb, preferred_element_type=jnp.float32) + bm_ref[...]
    )
    lv_ref[...] = (
        jnp.dot(wlt_ref[...], hb, preferred_element_type=jnp.float32) + bl_ref[...]
    )


@functools.lru_cache(maxsize=None)
def _make_tc_linear_t(B, D, O, grid):
    blk = B // grid
    return pl.pallas_call(
        _tc_linear_t_body,
        grid=(grid,),
        in_specs=[
            pl.BlockSpec((D, blk), lambda i: (0, i)),
            pl.BlockSpec((O, D), lambda i: (0, 0)),
            pl.BlockSpec((O, 1), lambda i: (0, 0)),
            pl.BlockSpec((O, D), lambda i: (0, 0)),
            pl.BlockSpec((O, 1), lambda i: (0, 0)),
        ],
        out_specs=[
            pl.BlockSpec((O, blk), lambda i: (0, i)),
            pl.BlockSpec((O, blk), lambda i: (0, i)),
        ],
        out_shape=[
            jax.ShapeDtypeStruct((O, B), jnp.float32),
            jax.ShapeDtypeStruct((O, B), jnp.float32),
        ],
    )


def kernel(x, table, W_mu, b_mu, W_logvar, b_logvar):
    B = x.shape[0]
    V, D = table.shape
    O = W_mu.shape[1]
    h_t = _make_sc_gather_t(B, V, D)(table.T, x)
    mu_t, lv_t = _make_tc_linear_t(B, D, O, 8)(
        h_t, W_mu.T, b_mu.reshape(O, 1), W_logvar.T, b_logvar.reshape(O, 1)
    )
    return (mu_t.T, lv_t.T)
